# baseline (speedup 1.0000x reference)
import functools

import numpy as np

import jax
import jax.numpy as jnp
from jax.experimental import pallas as pl
from jax.experimental.pallas import tpu as pltpu

_QTILE = 128
_NSTREAM = 8
_NEG = -1e30


def _local_bias() -> np.ndarray:
    j = np.arange(_QTILE)[:, None] // 16
    k = np.arange(_QTILE)[None, :] // 16
    valid = ((j // 4 == k // 4) & (k <= j)) | ((k == 3) & (j >= 4))
    return np.where(valid, 0.0, _NEG).astype(np.float32)


def _stripe_bias(t0: int, nt: int, scols: int) -> np.ndarray:
    t = np.arange(t0, t0 + nt)[:, None]
    col = np.arange(scols)[None, :]
    return np.where(col < 32 * t, 0.0, _NEG).astype(np.float32)[:, None, :]


def _make_body(t0: int, scols: int):
    nstripe = scols // 16

    def _attn_body(bl_ref, bs_ref, q_ref, k_ref, v_ref, o_ref,
                   ks_ref, vs_ref):
        t = t0 + pl.program_id(1)

        @pl.when(pl.program_id(1) == 0)
        def _gather_stripes():
            for u in range(_NSTREAM):
                for kk in range(nstripe):
                    src = kk * 64 + 48
                    dst = kk * 16
                    ks_ref[u, dst:dst + 16, :] = k_ref[0, u, src:src + 16, :]
                    vs_ref[u, dst:dst + 16, :] = (
                        v_ref[0, u, src:src + 16, :].astype(jnp.bfloat16))

        scale = q_ref.shape[-1] ** -0.5
        bias_loc = bl_ref[...]
        bias_str = bs_ref[0]

        for u in range(_NSTREAM):
            q = q_ref[0, u] * scale

            k_loc = k_ref[0, u, pl.ds(t * _QTILE, _QTILE), :]
            s_loc = jax.lax.dot_general(
                q, k_loc, (((1,), (1,)), ((), ())),
                preferred_element_type=jnp.float32) + bias_loc

            s_str = jax.lax.dot_general(
                q, ks_ref[u], (((1,), (1,)), ((), ())),
                preferred_element_type=jnp.float32) + bias_str

            m = jnp.maximum(jnp.max(s_loc, axis=1, keepdims=True),
                            jnp.max(s_str, axis=1, keepdims=True))
            e_loc = jnp.exp(s_loc - m)
            e_str = jnp.exp(s_str - m)
            inv = 1.0 / (jnp.sum(e_loc, axis=1, keepdims=True)
                         + jnp.sum(e_str, axis=1, keepdims=True))

            v_loc = (v_ref[0, u, pl.ds(t * _QTILE, _QTILE), :]
                     .astype(jnp.bfloat16))
            out = jax.lax.dot_general(
                e_str.astype(jnp.bfloat16), vs_ref[u],
                (((1,), (0,)), ((), ())),
                preferred_element_type=jnp.float32)
            out += jax.lax.dot_general(
                e_loc.astype(jnp.bfloat16), v_loc,
                (((1,), (0,)), ((), ())),
                preferred_element_type=jnp.float32)
            o_ref[0, u] = out * inv

    return _attn_body


def _run_span(q4, k4, v4, t0: int, nt: int, scols: int):
    g, nstream, s, dh = q4.shape
    kvrows = (t0 + nt) * _QTILE
    bias_loc = jnp.asarray(_local_bias())
    bias_str = jnp.asarray(_stripe_bias(t0, nt, scols))

    return pl.pallas_call(
        _make_body(t0, scols),
        grid=(g, nt),
        in_specs=[
            pl.BlockSpec((_QTILE, _QTILE), lambda i, t: (0, 0)),
            pl.BlockSpec((1, 1, scols), lambda i, t: (t, 0, 0)),
            pl.BlockSpec((1, nstream, _QTILE, dh),
                         lambda i, t, t0=t0: (i, 0, t0 + t, 0)),
            pl.BlockSpec((1, nstream, kvrows, dh), lambda i, t: (i, 0, 0, 0)),
            pl.BlockSpec((1, nstream, kvrows, dh), lambda i, t: (i, 0, 0, 0)),
        ],
        out_specs=pl.BlockSpec((1, nstream, _QTILE, dh),
                               lambda i, t: (i, 0, t, 0)),
        out_shape=jax.ShapeDtypeStruct((g, nstream, nt * _QTILE, dh),
                                       jnp.float32),
        scratch_shapes=[
            pltpu.VMEM((nstream, scols, dh), jnp.float32),
            pltpu.VMEM((nstream, scols, dh), jnp.bfloat16),
        ],
        compiler_params=pltpu.CompilerParams(
            dimension_semantics=("parallel", "arbitrary")),
    )(bias_loc, bias_str, q4, k4, v4)


@functools.partial(jax.jit, static_argnames=())
def kernel(query, key, value, mask):
    del mask
    b, h, s, dh = query.shape
    bh = b * h
    g = bh // _NSTREAM
    ntiles = s // _QTILE
    q4 = query.reshape(g, _NSTREAM, s, dh)
    k4 = key.reshape(g, _NSTREAM, s, dh)
    v4 = value.reshape(g, _NSTREAM, s, dh)

    def _scols(t_max: int) -> int:
        return max(128, -(-32 * t_max // 128) * 128)

    half = ntiles // 2
    lo = _run_span(q4, k4, v4, 0, half, _scols(half - 1))
    hi = _run_span(q4, k4, v4, half, ntiles - half, _scols(ntiles - 1))
    out = jnp.concatenate([lo, hi], axis=2)
    return out.reshape(b, h, s, dh)


# speedup vs baseline: 1.0607x; 1.0607x over previous
import functools

import numpy as np

import jax
import jax.numpy as jnp
from jax.experimental import pallas as pl
from jax.experimental.pallas import tpu as pltpu

_QTILE = 128
_NSTREAM = 8
_NEG = -1e30


def _local_bias() -> np.ndarray:
    j = np.arange(_QTILE)[:, None] // 16
    k = np.arange(_QTILE)[None, :] // 16
    valid = ((j // 4 == k // 4) & (k <= j)) | ((k == 3) & (j >= 4))
    return np.where(valid, 0.0, _NEG).astype(np.float32)


def _stripe_bias(t0: int, nt: int, scols: int) -> np.ndarray:
    t = np.arange(t0, t0 + nt)[:, None]
    col = np.arange(scols)[None, :]
    return np.where(col < 32 * t, 0.0, _NEG).astype(np.float32)[:, None, :]


def _make_body(t0: int, scols: int):
    nstripe = scols // 16

    def _attn_body(bl_ref, bs_ref, q_ref, k_ref, v_ref, o_ref,
                   ks_ref, vs_ref):
        t = t0 + pl.program_id(1)

        @pl.when(pl.program_id(1) == 0)
        def _gather_stripes():
            for u in range(_NSTREAM):
                for kk in range(nstripe):
                    src = kk * 64 + 48
                    dst = kk * 16
                    ks_ref[u, dst:dst + 16, :] = (
                        k_ref[0, u, src:src + 16, :].astype(jnp.bfloat16))
                    vs_ref[u, dst:dst + 16, :] = (
                        v_ref[0, u, src:src + 16, :].astype(jnp.bfloat16))

        scale = q_ref.shape[-1] ** -0.5
        bias_loc = bl_ref[...]
        bias_str = bs_ref[0]

        for u in range(_NSTREAM):
            q = (q_ref[0, u] * scale).astype(jnp.bfloat16)

            k_loc = (k_ref[0, u, pl.ds(t * _QTILE, _QTILE), :]
                     .astype(jnp.bfloat16))
            s_loc = jax.lax.dot_general(
                q, k_loc, (((1,), (1,)), ((), ())),
                preferred_element_type=jnp.float32) + bias_loc

            s_str = jax.lax.dot_general(
                q, ks_ref[u], (((1,), (1,)), ((), ())),
                preferred_element_type=jnp.float32) + bias_str

            m = jnp.maximum(jnp.max(s_loc, axis=1, keepdims=True),
                            jnp.max(s_str, axis=1, keepdims=True))
            e_loc = jnp.exp(s_loc - m)
            e_str = jnp.exp(s_str - m)
            inv = 1.0 / (jnp.sum(e_loc, axis=1, keepdims=True)
                         + jnp.sum(e_str, axis=1, keepdims=True))

            v_loc = (v_ref[0, u, pl.ds(t * _QTILE, _QTILE), :]
                     .astype(jnp.bfloat16))
            out = jax.lax.dot_general(
                e_str.astype(jnp.bfloat16), vs_ref[u],
                (((1,), (0,)), ((), ())),
                preferred_element_type=jnp.float32)
            out += jax.lax.dot_general(
                e_loc.astype(jnp.bfloat16), v_loc,
                (((1,), (0,)), ((), ())),
                preferred_element_type=jnp.float32)
            o_ref[0, u] = out * inv

    return _attn_body


def _run_span(q4, k4, v4, t0: int, nt: int, scols: int):
    g, nstream, s, dh = q4.shape
    kvrows = (t0 + nt) * _QTILE
    bias_loc = jnp.asarray(_local_bias())
    bias_str = jnp.asarray(_stripe_bias(t0, nt, scols))

    return pl.pallas_call(
        _make_body(t0, scols),
        grid=(g, nt),
        in_specs=[
            pl.BlockSpec((_QTILE, _QTILE), lambda i, t: (0, 0)),
            pl.BlockSpec((1, 1, scols), lambda i, t: (t, 0, 0)),
            pl.BlockSpec((1, nstream, _QTILE, dh),
                         lambda i, t, t0=t0: (i, 0, t0 + t, 0)),
            pl.BlockSpec((1, nstream, kvrows, dh), lambda i, t: (i, 0, 0, 0)),
            pl.BlockSpec((1, nstream, kvrows, dh), lambda i, t: (i, 0, 0, 0)),
        ],
        out_specs=pl.BlockSpec((1, nstream, _QTILE, dh),
                               lambda i, t: (i, 0, t, 0)),
        out_shape=jax.ShapeDtypeStruct((g, nstream, nt * _QTILE, dh),
                                       jnp.float32),
        scratch_shapes=[
            pltpu.VMEM((nstream, scols, dh), jnp.bfloat16),
            pltpu.VMEM((nstream, scols, dh), jnp.bfloat16),
        ],
        compiler_params=pltpu.CompilerParams(
            dimension_semantics=("parallel", "arbitrary")),
    )(bias_loc, bias_str, q4, k4, v4)


@functools.partial(jax.jit, static_argnames=())
def kernel(query, key, value, mask):
    del mask
    b, h, s, dh = query.shape
    bh = b * h
    g = bh // _NSTREAM
    ntiles = s // _QTILE
    q4 = query.reshape(g, _NSTREAM, s, dh)
    k4 = key.reshape(g, _NSTREAM, s, dh)
    v4 = value.reshape(g, _NSTREAM, s, dh)

    scols = max(128, -(-32 * (ntiles - 1) // 128) * 128)
    out = _run_span(q4, k4, v4, 0, ntiles, scols)
    return out.reshape(b, h, s, dh)


# speedup vs baseline: 1.0824x; 1.0205x over previous
import functools

import numpy as np

import jax
import jax.numpy as jnp
from jax.experimental import pallas as pl
from jax.experimental.pallas import tpu as pltpu

_QTILE = 128
_NSTREAM = 8
_NEG = -1e30


def _local_bias() -> np.ndarray:
    j = np.arange(_QTILE)[:, None] // 16
    k = np.arange(_QTILE)[None, :] // 16
    valid = ((j // 4 == k // 4) & (k <= j)) | ((k == 3) & (j >= 4))
    return np.where(valid, 0.0, _NEG).astype(np.float32)


def _stripe_bias(t0: int, nt: int, scols: int) -> np.ndarray:
    t = np.arange(t0, t0 + nt)[:, None]
    col = np.arange(scols)[None, :]
    return np.where(col < 32 * t, 0.0, _NEG).astype(np.float32)[:, None, :]


def _make_body(t0: int, scols: int):
    nstripe = scols // 16

    def _attn_body(bl_ref, bs_ref, q_ref, k_ref, v_ref, o_ref,
                   ks_ref, vs_ref):
        t = t0 + pl.program_id(1)

        @pl.when(pl.program_id(1) == 0)
        def _gather_stripes():
            for u in range(_NSTREAM):
                for kk in range(nstripe):
                    src = kk * 64 + 48
                    dst = kk * 16
                    ks_ref[u, dst:dst + 16, :] = k_ref[0, u, src:src + 16, :]
                    vs_ref[u, dst:dst + 16, :] = (
                        v_ref[0, u, src:src + 16, :].astype(jnp.bfloat16))

        scale = q_ref.shape[-1] ** -0.5
        bias_loc = bl_ref[...]
        bias_str = bs_ref[0]

        for u in range(_NSTREAM):
            q = q_ref[0, u] * scale

            k_loc = k_ref[0, u, pl.ds(t * _QTILE, _QTILE), :]
            s_loc = jax.lax.dot_general(
                q, k_loc, (((1,), (1,)), ((), ())),
                preferred_element_type=jnp.float32) + bias_loc

            s_str = jax.lax.dot_general(
                q, ks_ref[u], (((1,), (1,)), ((), ())),
                preferred_element_type=jnp.float32) + bias_str

            m = jnp.maximum(jnp.max(s_loc, axis=1, keepdims=True),
                            jnp.max(s_str, axis=1, keepdims=True))
            e_loc = jnp.exp(s_loc - m)
            e_str = jnp.exp(s_str - m)
            inv = 1.0 / (jnp.sum(e_loc, axis=1, keepdims=True)
                         + jnp.sum(e_str, axis=1, keepdims=True))

            v_loc = (v_ref[0, u, pl.ds(t * _QTILE, _QTILE), :]
                     .astype(jnp.bfloat16))
            out = jax.lax.dot_general(
                e_str.astype(jnp.bfloat16), vs_ref[u],
                (((1,), (0,)), ((), ())),
                preferred_element_type=jnp.float32)
            out += jax.lax.dot_general(
                e_loc.astype(jnp.bfloat16), v_loc,
                (((1,), (0,)), ((), ())),
                preferred_element_type=jnp.float32)
            o_ref[0, u] = out * inv

    return _attn_body


def _run_span(q4, k4, v4, t0: int, nt: int, scols: int):
    g, nstream, s, dh = q4.shape
    kvrows = (t0 + nt) * _QTILE
    bias_loc = jnp.asarray(_local_bias())
    bias_str = jnp.asarray(_stripe_bias(t0, nt, scols))

    return pl.pallas_call(
        _make_body(t0, scols),
        grid=(g, nt),
        in_specs=[
            pl.BlockSpec((_QTILE, _QTILE), lambda i, t: (0, 0)),
            pl.BlockSpec((1, 1, scols), lambda i, t: (t, 0, 0)),
            pl.BlockSpec((1, nstream, _QTILE, dh),
                         lambda i, t, t0=t0: (i, 0, t0 + t, 0)),
            pl.BlockSpec((1, nstream, kvrows, dh), lambda i, t: (i, 0, 0, 0)),
            pl.BlockSpec((1, nstream, kvrows, dh), lambda i, t: (i, 0, 0, 0)),
        ],
        out_specs=pl.BlockSpec((1, nstream, _QTILE, dh),
                               lambda i, t: (i, 0, t, 0)),
        out_shape=jax.ShapeDtypeStruct((g, nstream, nt * _QTILE, dh),
                                       jnp.float32),
        scratch_shapes=[
            pltpu.VMEM((nstream, scols, dh), jnp.float32),
            pltpu.VMEM((nstream, scols, dh), jnp.bfloat16),
        ],
        compiler_params=pltpu.CompilerParams(
            dimension_semantics=("parallel", "arbitrary")),
    )(bias_loc, bias_str, q4, k4, v4)


@functools.partial(jax.jit, static_argnames=())
def kernel(query, key, value, mask):
    del mask
    b, h, s, dh = query.shape
    bh = b * h
    g = bh // _NSTREAM
    ntiles = s // _QTILE
    q4 = query.reshape(g, _NSTREAM, s, dh)
    k4 = key.reshape(g, _NSTREAM, s, dh)
    v4 = value.reshape(g, _NSTREAM, s, dh)

    scols = max(128, -(-32 * (ntiles - 1) // 128) * 128)
    out = _run_span(q4, k4, v4, 0, ntiles, scols)
    return out.reshape(b, h, s, dh)


# speedup vs baseline: 1.1359x; 1.0494x over previous
import functools

import numpy as np

import jax
import jax.numpy as jnp
from jax.experimental import pallas as pl
from jax.experimental.pallas import tpu as pltpu

_QTILE = 128
_NSTREAM = 8
_NEG = -1e30


def _local_bias() -> np.ndarray:
    j = np.arange(_QTILE)[:, None] // 16
    k = np.arange(_QTILE)[None, :] // 16
    valid = ((j // 4 == k // 4) & (k <= j)) | ((k == 3) & (j >= 4))
    return np.where(valid, 0.0, _NEG).astype(np.float32)


def _stripe_bias(nt: int, scols: int) -> np.ndarray:
    t = np.arange(nt)[:, None]
    col = np.arange(scols)[None, :]
    return np.where(col < 32 * t, 0.0, _NEG).astype(np.float32)[:, None, :]


def _make_body(scols: int):
    nstripe = scols // 16

    def _attn_body(bl_ref, bs_ref, q_ref, k_hbm, v_hbm, o_ref,
                   kbuf, vbuf, ks_ref, vs_ref, sem):
        i = pl.program_id(0)
        t = pl.program_id(1)
        ni = pl.num_programs(0)
        slot = jax.lax.rem(i, 2)

        def _start(src_i, dst_slot):
            pltpu.make_async_copy(
                k_hbm.at[src_i], kbuf.at[dst_slot], sem.at[dst_slot, 0]
            ).start()
            pltpu.make_async_copy(
                v_hbm.at[src_i], vbuf.at[dst_slot], sem.at[dst_slot, 1]
            ).start()

        @pl.when((i == 0) & (t == 0))
        def _bootstrap():
            _start(0, 0)

        @pl.when(t == 0)
        def _land_and_prefetch():
            pltpu.make_async_copy(
                k_hbm.at[i], kbuf.at[slot], sem.at[slot, 0]).wait()
            pltpu.make_async_copy(
                v_hbm.at[i], vbuf.at[slot], sem.at[slot, 1]).wait()

            @pl.when(i + 1 < ni)
            def _prefetch_next():
                _start(i + 1, 1 - slot)

            for u in range(_NSTREAM):
                for kk in range(nstripe):
                    src = kk * 64 + 48
                    dst = kk * 16
                    ks_ref[u, dst:dst + 16, :] = kbuf[slot, u, src:src + 16, :]
                    vs_ref[u, dst:dst + 16, :] = (
                        vbuf[slot, u, src:src + 16, :].astype(jnp.bfloat16))

        scale = q_ref.shape[-1] ** -0.5
        bias_loc = bl_ref[...]
        bias_str = bs_ref[0]

        for u in range(_NSTREAM):
            q = q_ref[0, u] * scale

            k_loc = kbuf[slot, u, pl.ds(t * _QTILE, _QTILE), :]
            s_loc = jax.lax.dot_general(
                q, k_loc, (((1,), (1,)), ((), ())),
                preferred_element_type=jnp.float32) + bias_loc

            s_str = jax.lax.dot_general(
                q, ks_ref[u], (((1,), (1,)), ((), ())),
                preferred_element_type=jnp.float32) + bias_str

            m = jnp.maximum(jnp.max(s_loc, axis=1, keepdims=True),
                            jnp.max(s_str, axis=1, keepdims=True))
            e_loc = jnp.exp(s_loc - m)
            e_str = jnp.exp(s_str - m)
            inv = 1.0 / (jnp.sum(e_loc, axis=1, keepdims=True)
                         + jnp.sum(e_str, axis=1, keepdims=True))

            v_loc = (vbuf[slot, u, pl.ds(t * _QTILE, _QTILE), :]
                     .astype(jnp.bfloat16))
            out = jax.lax.dot_general(
                e_str.astype(jnp.bfloat16), vs_ref[u],
                (((1,), (0,)), ((), ())),
                preferred_element_type=jnp.float32)
            out += jax.lax.dot_general(
                e_loc.astype(jnp.bfloat16), v_loc,
                (((1,), (0,)), ((), ())),
                preferred_element_type=jnp.float32)
            o_ref[0, u] = out * inv

    return _attn_body


@functools.partial(jax.jit, static_argnames=())
def kernel(query, key, value, mask):
    del mask
    b, h, s, dh = query.shape
    bh = b * h
    g = bh // _NSTREAM
    ntiles = s // _QTILE
    q4 = query.reshape(g, _NSTREAM, s, dh)
    k4 = key.reshape(g, _NSTREAM, s, dh)
    v4 = value.reshape(g, _NSTREAM, s, dh)

    scols = max(128, -(-32 * (ntiles - 1) // 128) * 128)
    bias_loc = jnp.asarray(_local_bias())
    bias_str = jnp.asarray(_stripe_bias(ntiles, scols))

    out = pl.pallas_call(
        _make_body(scols),
        grid=(g, ntiles),
        in_specs=[
            pl.BlockSpec((_QTILE, _QTILE), lambda i, t: (0, 0)),
            pl.BlockSpec((1, 1, scols), lambda i, t: (t, 0, 0)),
            pl.BlockSpec((1, _NSTREAM, _QTILE, dh),
                         lambda i, t: (i, 0, t, 0)),
            pl.BlockSpec(memory_space=pltpu.MemorySpace.HBM),
            pl.BlockSpec(memory_space=pltpu.MemorySpace.HBM),
        ],
        out_specs=pl.BlockSpec((1, _NSTREAM, _QTILE, dh),
                               lambda i, t: (i, 0, t, 0)),
        out_shape=jax.ShapeDtypeStruct((g, _NSTREAM, s, dh), jnp.float32),
        scratch_shapes=[
            pltpu.VMEM((2, _NSTREAM, s, dh), jnp.float32),
            pltpu.VMEM((2, _NSTREAM, s, dh), jnp.float32),
            pltpu.VMEM((_NSTREAM, scols, dh), jnp.float32),
            pltpu.VMEM((_NSTREAM, scols, dh), jnp.bfloat16),
            pltpu.SemaphoreType.DMA((2, 2)),
        ],
        compiler_params=pltpu.CompilerParams(
            dimension_semantics=("arbitrary", "arbitrary")),
    )(bias_loc, bias_str, q4, k4, v4)
    return out.reshape(b, h, s, dh)


# speedup vs baseline: 1.4593x; 1.2847x over previous
import functools

import numpy as np

import jax
import jax.numpy as jnp
from jax.experimental import pallas as pl
from jax.experimental.pallas import tpu as pltpu

_QTILE = 128
_NSTRIPE = 32
_SCOLS = _NSTRIPE * 16
_NEG = -1e30


def _round128(n: int) -> int:
    return -(-n // 128) * 128


def _local_bias() -> np.ndarray:
    j = np.arange(_QTILE)[:, None] // 16
    k = np.arange(_QTILE)[None, :] // 16
    valid = ((j // 4 == k // 4) & (k <= j)) | ((k == 3) & (j >= 4))
    return np.where(valid, 0.0, _NEG).astype(np.float32)


def _stripe_bias(nt: int) -> np.ndarray:
    t = np.arange(nt)[:, None]
    col = np.arange(_SCOLS)[None, :]
    return np.where(col < 32 * t, 0.0, _NEG).astype(np.float32)


def _make_body(ntiles: int):
    def _attn_body(bl_ref, bs_ref, q_ref, k_ref, v_ref, o_ref,
                   ks_ref, vs_ref):
        for kk in range(_NSTRIPE):
            src = kk * 64 + 48
            dst = kk * 16
            ks_ref[dst:dst + 16, :] = k_ref[0, src:src + 16, :]
            vs_ref[dst:dst + 16, :] = (
                v_ref[0, src:src + 16, :].astype(jnp.bfloat16))

        scale = q_ref.shape[-1] ** -0.5
        bias_loc = bl_ref[...]

        for t in range(ntiles):
            lo = t * _QTILE
            q = q_ref[0, lo:lo + _QTILE, :] * scale

            s_loc = jax.lax.dot_general(
                q, k_ref[0, lo:lo + _QTILE, :], (((1,), (1,)), ((), ())),
                preferred_element_type=jnp.float32) + bias_loc
            m = jnp.max(s_loc, axis=1, keepdims=True)

            w = _round128(32 * t)
            if w:
                s_str = jax.lax.dot_general(
                    q, ks_ref[0:w, :], (((1,), (1,)), ((), ())),
                    preferred_element_type=jnp.float32) + bs_ref[t:t + 1, 0:w]
                m = jnp.maximum(m, jnp.max(s_str, axis=1, keepdims=True))

            e_loc = jnp.exp(s_loc - m)
            denom = jnp.sum(e_loc, axis=1, keepdims=True)
            out = jax.lax.dot_general(
                e_loc.astype(jnp.bfloat16),
                v_ref[0, lo:lo + _QTILE, :].astype(jnp.bfloat16),
                (((1,), (0,)), ((), ())),
                preferred_element_type=jnp.float32)
            if w:
                e_str = jnp.exp(s_str - m)
                denom += jnp.sum(e_str, axis=1, keepdims=True)
                out += jax.lax.dot_general(
                    e_str.astype(jnp.bfloat16), vs_ref[0:w, :],
                    (((1,), (0,)), ((), ())),
                    preferred_element_type=jnp.float32)

            o_ref[0, lo:lo + _QTILE, :] = out * (1.0 / denom)

    return _attn_body


@functools.partial(jax.jit, static_argnames=())
def kernel(query, key, value, mask):
    del mask
    b, h, s, dh = query.shape
    bh = b * h
    ntiles = s // _QTILE
    q3 = query.reshape(bh, s, dh)
    k3 = key.reshape(bh, s, dh)
    v3 = value.reshape(bh, s, dh)
    bias_loc = jnp.asarray(_local_bias())
    bias_str = jnp.asarray(_stripe_bias(ntiles))

    out = pl.pallas_call(
        _make_body(ntiles),
        grid=(bh,),
        in_specs=[
            pl.BlockSpec((_QTILE, _QTILE), lambda i: (0, 0)),
            pl.BlockSpec((ntiles, _SCOLS), lambda i: (0, 0)),
            pl.BlockSpec((1, s, dh), lambda i: (i, 0, 0)),
            pl.BlockSpec((1, s, dh), lambda i: (i, 0, 0)),
            pl.BlockSpec((1, s, dh), lambda i: (i, 0, 0)),
        ],
        out_specs=pl.BlockSpec((1, s, dh), lambda i: (i, 0, 0)),
        out_shape=jax.ShapeDtypeStruct((bh, s, dh), jnp.float32),
        scratch_shapes=[
            pltpu.VMEM((_SCOLS, dh), jnp.float32),
            pltpu.VMEM((_SCOLS, dh), jnp.bfloat16),
        ],
        compiler_params=pltpu.CompilerParams(
            dimension_semantics=("arbitrary",)),
    )(bias_loc, bias_str, q3, k3, v3)
    return out.reshape(b, h, s, dh)


# speedup vs baseline: 2.4186x; 1.6574x over previous
import functools

import numpy as np

import jax
import jax.numpy as jnp
from jax.experimental import pallas as pl
from jax.experimental.pallas import tpu as pltpu

_QTILE = 128
_NSTRIPE = 32
_SCOLS = _NSTRIPE * 16
_NEG = -1e30


def _round128(n: int) -> int:
    return -(-n // 128) * 128


def _local_bias() -> np.ndarray:
    j = np.arange(_QTILE)[:, None] // 16
    k = np.arange(_QTILE)[None, :] // 16
    valid = ((j // 4 == k // 4) & (k <= j)) | ((k == 3) & (j >= 4))
    return np.where(valid, 0.0, _NEG).astype(np.float32)


def _stripe_bias(nt: int) -> np.ndarray:
    t = np.arange(nt)[:, None]
    col = np.arange(_SCOLS)[None, :]
    return np.where(col < 32 * t, 0.0, _NEG).astype(np.float32)


def _make_body(ntiles: int):
    def _attn_body(bl_ref, bs_ref, q_ref, k_ref, v_ref, o_ref,
                   ks_ref, vs_ref):
        for kk in range(_NSTRIPE):
            src = kk * 64 + 48
            dst = kk * 16
            ks_ref[dst:dst + 16, :] = k_ref[0, src:src + 16, :]
            vs_ref[dst:dst + 16, :] = (
                v_ref[0, src:src + 16, :].astype(jnp.bfloat16))

        scale = q_ref.shape[-1] ** -0.5
        bias_loc = bl_ref[...]

        def _qk(t):
            lo = t * _QTILE
            q = q_ref[0, lo:lo + _QTILE, :] * scale
            s_loc = jax.lax.dot_general(
                q, k_ref[0, lo:lo + _QTILE, :], (((1,), (1,)), ((), ())),
                preferred_element_type=jnp.float32) + bias_loc
            w = _round128(32 * t)
            s_str = None
            if w:
                s_str = jax.lax.dot_general(
                    q, ks_ref[0:w, :], (((1,), (1,)), ((), ())),
                    preferred_element_type=jnp.float32) + bs_ref[t:t + 1, 0:w]
            return s_loc, s_str, w

        def _sv(t, scores):
            s_loc, s_str, w = scores
            lo = t * _QTILE
            m = jnp.max(s_loc, axis=1, keepdims=True)
            if w:
                m = jnp.maximum(m, jnp.max(s_str, axis=1, keepdims=True))
            e_loc = jnp.exp(s_loc - m)
            denom = jnp.sum(e_loc, axis=1, keepdims=True)
            out = jax.lax.dot_general(
                e_loc.astype(jnp.bfloat16),
                v_ref[0, lo:lo + _QTILE, :].astype(jnp.bfloat16),
                (((1,), (0,)), ((), ())),
                preferred_element_type=jnp.float32)
            if w:
                e_str = jnp.exp(s_str - m)
                denom += jnp.sum(e_str, axis=1, keepdims=True)
                out += jax.lax.dot_general(
                    e_str.astype(jnp.bfloat16), vs_ref[0:w, :],
                    (((1,), (0,)), ((), ())),
                    preferred_element_type=jnp.float32)
            o_ref[0, lo:lo + _QTILE, :] = out * (1.0 / denom)

        scores = _qk(0)
        for t in range(ntiles):
            nxt = _qk(t + 1) if t + 1 < ntiles else None
            _sv(t, scores)
            scores = nxt

    return _attn_body


@functools.partial(jax.jit, static_argnames=())
def kernel(query, key, value, mask):
    del mask
    b, h, s, dh = query.shape
    bh = b * h
    ntiles = s // _QTILE
    q3 = query.reshape(bh, s, dh)
    k3 = key.reshape(bh, s, dh)
    v3 = value.reshape(bh, s, dh)
    bias_loc = jnp.asarray(_local_bias())
    bias_str = jnp.asarray(_stripe_bias(ntiles))

    out = pl.pallas_call(
        _make_body(ntiles),
        grid=(bh,),
        in_specs=[
            pl.BlockSpec((_QTILE, _QTILE), lambda i: (0, 0)),
            pl.BlockSpec((ntiles, _SCOLS), lambda i: (0, 0)),
            pl.BlockSpec((1, s, dh), lambda i: (i, 0, 0)),
            pl.BlockSpec((1, s, dh), lambda i: (i, 0, 0)),
            pl.BlockSpec((1, s, dh), lambda i: (i, 0, 0)),
        ],
        out_specs=pl.BlockSpec((1, s, dh), lambda i: (i, 0, 0)),
        out_shape=jax.ShapeDtypeStruct((bh, s, dh), jnp.float32),
        scratch_shapes=[
            pltpu.VMEM((_SCOLS, dh), jnp.float32),
            pltpu.VMEM((_SCOLS, dh), jnp.bfloat16),
        ],
        compiler_params=pltpu.CompilerParams(
            dimension_semantics=("arbitrary",)),
    )(bias_loc, bias_str, q3, k3, v3)
    return out.reshape(b, h, s, dh)
